# baseline (device time: 17274 ns/iter reference)
import jax
import jax.numpy as jnp
from jax import lax
from jax.experimental import pallas as pl
from jax.experimental.pallas import tpu as pltpu

Z = 4
X = 2


def kernel(x):
    m, n = x.shape
    blk = n // Z
    half = m // X

    def body(x_ref, out_ref, zsend, zrecv, xsend, xrecv):
        mx = lax.axis_index("x")
        my = lax.axis_index("y")
        mz = lax.axis_index("z")
        px = 1 - mx

        barrier_sem = pltpu.get_barrier_semaphore()
        for r in range(1, Z):
            pl.semaphore_signal(
                barrier_sem, inc=1,
                device_id=(mx, my, (mz + r) % Z),
                device_id_type=pl.DeviceIdType.MESH,
            )
        pl.semaphore_signal(
            barrier_sem, inc=1,
            device_id=(px, my, mz),
            device_id_type=pl.DeviceIdType.MESH,
        )
        pl.semaphore_wait(barrier_sem, Z)

        zs = []
        for r in range(1, Z):
            q = (mz + r) % Z
            rdma = pltpu.make_async_remote_copy(
                src_ref=x_ref.at[pl.ds(mx * half, half), pl.ds(q * blk, blk)],
                dst_ref=out_ref.at[pl.ds(mz * m + mx * half, half), :],
                send_sem=zsend.at[r - 1],
                recv_sem=zrecv.at[r - 1],
                device_id=(mx, my, q),
                device_id_type=pl.DeviceIdType.MESH,
            )
            rdma.start()
            zs.append(rdma)

        out_ref[pl.ds(mz * m, m), :] = x_ref[:, pl.ds(mz * blk, blk)]

        xs = []
        for r in range(1, Z):
            p = (mz - r) % Z
            zs[r - 1].wait_recv()
            rows = pl.ds(p * m + mx * half, half)
            rdma = pltpu.make_async_remote_copy(
                src_ref=out_ref.at[rows, :],
                dst_ref=out_ref.at[rows, :],
                send_sem=xsend.at[r - 1],
                recv_sem=xrecv.at[r - 1],
                device_id=(px, my, mz),
                device_id_type=pl.DeviceIdType.MESH,
            )
            rdma.start()
            xs.append(rdma)

        for r in range(1, Z):
            xs[r - 1].wait()
            zs[r - 1].wait_send()

    out_shape = jax.ShapeDtypeStruct((Z * m, blk), x.dtype)
    return pl.pallas_call(
        body,
        out_shape=out_shape,
        in_specs=[pl.BlockSpec(memory_space=pltpu.VMEM)],
        out_specs=pl.BlockSpec(memory_space=pltpu.VMEM),
        scratch_shapes=[
            pltpu.SemaphoreType.DMA((Z - 1,)),
            pltpu.SemaphoreType.DMA((Z - 1,)),
            pltpu.SemaphoreType.DMA((Z - 1,)),
            pltpu.SemaphoreType.DMA((Z - 1,)),
        ],
        compiler_params=pltpu.CompilerParams(collective_id=0),
    )(x)
